# Initial kernel scaffold; baseline (speedup 1.0000x reference)
import jax
import jax.numpy as jnp
from jax.experimental import pallas as pl


def kernel(h, x, edge_index, edge_dist, We1, be1, We2, be2, Wn1, bn1, Wn2, bn2, Wc1, bc1, Wc2):
    raise NotImplementedError("write your pallas kernel here")


# speedup vs baseline: 1.7868x; 1.7868x over previous
import functools

import jax
import jax.numpy as jnp
from jax import lax
from jax.experimental import pallas as pl
from jax.experimental.pallas import tpu as pltpu
from jax.experimental.pallas import tpu_sc as plsc

NC = 2
NS = 16
NW = NC * NS
C = 128
L = 16


def _gather_kernel(cpt, hpad, xt4, etab, srcp, dstp, hs, hd, dir4,
                   xt4v, etv, isrc, idst, bsh, bdh, dbuf, sem):
    wid = lax.axis_index("s") * NC + lax.axis_index("c")
    base0 = wid * (C * cpt)
    pltpu.sync_copy(xt4, xt4v)
    pltpu.sync_copy(etab, etv)

    def body(j, _):
        base = base0 + j * C
        pltpu.sync_copy(srcp.at[pl.ds(base, C)], isrc)
        pltpu.sync_copy(dstp.at[pl.ds(base, C)], idst)
        c1 = pltpu.async_copy(hpad.at[isrc], bsh, sem)
        c2 = pltpu.async_copy(hpad.at[idst], bdh, sem)

        def xbody(k, _):
            ev = etv[pl.ds(L * k, L)]
            comp = etv[pl.ds((C // 4) * L, L)]
            s4 = plsc.load_gather(isrc, [ev]) * 4 + comp
            d4 = plsc.load_gather(idst, [ev]) * 4 + comp
            dv = plsc.load_gather(xt4v, [s4]) - plsc.load_gather(xt4v, [d4])
            dbuf[pl.ds(L * k, L)] = dv
            return ()

        lax.fori_loop(0, C // 4, xbody, ())
        c1.wait()
        c2.wait()
        pltpu.sync_copy(bsh, hs.at[pl.ds(base, C)])
        pltpu.sync_copy(bdh, hd.at[pl.ds(base, C)])
        pltpu.sync_copy(dbuf, dir4.at[pl.ds(base * 4, C * 4)])
        return ()

    lax.fori_loop(0, cpt, body, ())


def _scatter_kernel(cpt2, npad, m, cuf, dstp, zh, ph, pxt,
                    acc_h, accx, bm, bcf, idst, sem):
    cid = lax.axis_index("c")
    sid = lax.axis_index("s")
    wid = sid * NC + cid

    rpt = npad // NS
    pltpu.sync_copy(zh.at[pl.ds(sid * rpt, rpt)],
                    acc_h.at[pl.ds(sid * rpt, rpt)])

    def zbody(i, _):
        accx[pl.ds(L * i, L)] = jnp.zeros((L,), jnp.float32)
        return ()

    lax.fori_loop(0, npad * 3 // L, zbody, ())
    plsc.subcore_barrier()

    base0 = wid * (C * cpt2)

    def body(j, _):
        base = base0 + j * C
        pltpu.sync_copy(dstp.at[pl.ds(base, C)], idst)
        pltpu.sync_copy(m.at[pl.ds(base, C)], bm)
        cpy = pltpu.async_copy(bm, acc_h.at[idst], sem, add=True)
        pltpu.sync_copy(cuf.at[pl.ds(base * L, C * L)], bcf)

        def xbody(e, _):
            ev = jnp.full((L,), e, jnp.int32)
            dvec = plsc.load_gather(idst, [ev])
            addr = dvec * 3 + lax.iota(jnp.int32, L)
            vals = bcf[pl.ds(L * e, L)]
            plsc.addupdate_scatter(accx, [addr], vals)
            return ()

        lax.fori_loop(0, C, xbody, ())
        cpy.wait()
        return ()

    lax.fori_loop(0, cpt2, body, ())
    plsc.subcore_barrier()

    @pl.when(sid == 0)
    def _():
        pltpu.sync_copy(acc_h, ph.at[cid])

    pltpu.sync_copy(accx, pxt.at[wid])


def _edge_kernel(hs_ref, hd_ref, dir_ref, dist_ref,
                 we1_ref, be1_ref, we2_ref, be2_ref,
                 w1s_ref, w1d_ref, w1e_ref, b1_ref,
                 wn2_ref, bn2_ref, wc2_ref,
                 m_ref, cu_ref):
    f32 = jnp.float32
    hp = jax.lax.Precision.HIGHEST
    d = dist_ref[0, 0, :][:, None]
    ea = d * we1_ref[...] + be1_ref[...]
    ea = ea * jax.nn.sigmoid(ea)
    ea = jnp.dot(ea, we2_ref[...], precision=hp, preferred_element_type=f32)
    ea = ea + be2_ref[...]
    z = jnp.dot(hs_ref[...], w1s_ref[...], precision=hp, preferred_element_type=f32)
    z = z + jnp.dot(hd_ref[...], w1d_ref[...], precision=hp, preferred_element_type=f32)
    z = z + jnp.dot(ea, w1e_ref[...], precision=hp, preferred_element_type=f32)
    z = z + b1_ref[...]
    u = z * jax.nn.sigmoid(z)
    un = u[:, :128]
    uc = u[:, 128:]
    m = jnp.dot(un, wn2_ref[...], precision=hp, preferred_element_type=f32)
    m_ref[...] = m + bn2_ref[...]
    cw = jnp.sum(uc * wc2_ref[...], axis=1, keepdims=True)
    dirv = dir_ref[...]
    n2 = jnp.sum(dirv * dirv, axis=1, keepdims=True)
    dl = jnp.maximum(jnp.sqrt(n2), 1e-8)
    cu4 = (cw / dl) * dirv
    cu_ref[...] = jnp.concatenate(
        [cu4, jnp.zeros((cu4.shape[0], 12), f32)], axis=1)


def _combine_kernel(hpad_ref, xpp_ref, ph_ref, px_ref, ho_ref, xo_ref):
    ho_ref[...] = hpad_ref[...] + ph_ref[0] + ph_ref[1]
    xo_ref[...] = xpp_ref[...] + jnp.sum(px_ref[...], axis=0)


def kernel(h, x, edge_index, edge_dist, We1, be1, We2, be2,
           Wn1, bn1, Wn2, bn2, Wc1, bc1, Wc2):
    f32 = jnp.float32
    i32 = jnp.int32
    N, D = h.shape
    E = edge_index.shape[1]

    cpt = -(-E // (NW * C))
    e_pad = NW * C * cpt
    cpt2 = e_pad // (NW * C)
    npad = -(-(N + 1) // 128) * 128

    src = jnp.pad(edge_index[0], (0, e_pad - E))
    dst = jnp.pad(edge_index[1], (0, e_pad - E),
                  constant_values=N)
    distp = jnp.pad(edge_dist, (0, e_pad - E))

    hpad = jnp.pad(h, ((0, npad - N), (0, 0)))
    xt4 = jnp.pad(x, ((0, npad - N), (0, 4 - x.shape[1]))).reshape(-1)
    xpp3 = x if x.shape[1] == 3 else jnp.pad(x, ((0, 0), (0, 3 - x.shape[1])))
    xpp3 = jnp.pad(xpp3, ((0, npad - N), (0, 0)))

    p = jnp.arange(C // 4 * L, dtype=i32)
    etab = (p // L * 4 + p % L // 4).reshape(C // 4, L)
    etab = jnp.concatenate(
        [etab, jnp.tile(jnp.arange(4, dtype=i32), L // 4)[None, :]]).reshape(-1)

    W1 = jnp.concatenate([Wn1, Wc1], axis=1)
    W1s, W1d, W1e = W1[:D], W1[D:2 * D], W1[2 * D:]
    b1 = jnp.concatenate([bn1, bc1])[None, :]
    wc2 = Wc2[:, 0][None, :]

    mesh = plsc.VectorSubcoreMesh(core_axis_name="c", subcore_axis_name="s",
                                  num_cores=NC, num_subcores=NS)

    gather = pl.kernel(
        functools.partial(_gather_kernel, cpt),
        out_type=(
            jax.ShapeDtypeStruct((e_pad, D), f32),
            jax.ShapeDtypeStruct((e_pad, D), f32),
            jax.ShapeDtypeStruct((e_pad * 4,), f32),
        ),
        mesh=mesh,
        scratch_types=[
            pltpu.VMEM((npad * 4,), f32),
            pltpu.VMEM(((C // 4 + 1) * L,), i32),
            pltpu.VMEM((C,), i32),
            pltpu.VMEM((C,), i32),
            pltpu.VMEM((C, D), f32),
            pltpu.VMEM((C, D), f32),
            pltpu.VMEM((C * 4,), f32),
            pltpu.SemaphoreType.DMA,
        ],
        name="egnn_sc_gather",
        compiler_params=pltpu.CompilerParams(needs_layout_passes=False),
    )
    hs, hd, dir4 = gather(hpad, xt4, etab, src, dst)

    BE = 512
    grid = e_pad // BE
    dist3 = distp.reshape(grid, 1, BE)
    full = lambda shape: pl.BlockSpec(shape, lambda i: (0,) * len(shape))
    m, cu = pl.pallas_call(
        _edge_kernel,
        grid=(grid,),
        in_specs=[
            pl.BlockSpec((BE, D), lambda i: (i, 0)),
            pl.BlockSpec((BE, D), lambda i: (i, 0)),
            pl.BlockSpec((BE, 4), lambda i: (i, 0)),
            pl.BlockSpec((1, 1, BE), lambda i: (i, 0, 0)),
            full((1, 16)), full((1, 16)), full((16, 16)), full((1, 16)),
            full((D, 256)), full((D, 256)), full((16, 256)), full((1, 256)),
            full((D, D)), full((1, D)), full((1, D)),
        ],
        out_specs=[
            pl.BlockSpec((BE, D), lambda i: (i, 0)),
            pl.BlockSpec((BE, 16), lambda i: (i, 0)),
        ],
        out_shape=[
            jax.ShapeDtypeStruct((e_pad, D), f32),
            jax.ShapeDtypeStruct((e_pad, 16), f32),
        ],
        name="egnn_tc_edge_mlp",
    )(hs, hd, dir4.reshape(e_pad, 4), dist3,
      We1, be1[None, :], We2, be2[None, :],
      W1s, W1d, W1e, b1, Wn2, bn2[None, :], wc2)

    zh = jnp.zeros((npad, D), f32)
    scatter = pl.kernel(
        functools.partial(_scatter_kernel, cpt2, npad),
        out_type=(
            jax.ShapeDtypeStruct((NC, npad, D), f32),
            jax.ShapeDtypeStruct((NW, npad * 3), f32),
        ),
        mesh=mesh,
        scratch_types=[
            pltpu.VMEM_SHARED((npad, D), f32),
            pltpu.VMEM((npad * 3,), f32),
            pltpu.VMEM((C, D), f32),
            pltpu.VMEM((C * L,), f32),
            pltpu.VMEM((C,), i32),
            pltpu.SemaphoreType.DMA,
        ],
        name="egnn_sc_scatter",
        compiler_params=pltpu.CompilerParams(needs_layout_passes=False),
    )
    ph, pxt = scatter(m, cu.reshape(-1), dst, zh)

    RB = 128
    ho, xo = pl.pallas_call(
        _combine_kernel,
        grid=(npad // RB,),
        in_specs=[
            pl.BlockSpec((RB, D), lambda i: (i, 0)),
            pl.BlockSpec((RB, 3), lambda i: (i, 0)),
            pl.BlockSpec((NC, RB, D), lambda i: (0, i, 0)),
            pl.BlockSpec((NW, RB, 3), lambda i: (0, i, 0)),
        ],
        out_specs=[
            pl.BlockSpec((RB, D), lambda i: (i, 0)),
            pl.BlockSpec((RB, 3), lambda i: (i, 0)),
        ],
        out_shape=[
            jax.ShapeDtypeStruct((npad, D), f32),
            jax.ShapeDtypeStruct((npad, 3), f32),
        ],
        name="egnn_tc_combine",
    )(hpad, xpp3, ph, pxt.reshape(NW, npad, 3))

    return ho[:N], xo[:N]


# speedup vs baseline: 2.5127x; 1.4062x over previous
import functools

import jax
import jax.numpy as jnp
from jax import lax
from jax.experimental import pallas as pl
from jax.experimental.pallas import tpu as pltpu
from jax.experimental.pallas import tpu_sc as plsc

NC = 2
NS = 16
NW = NC * NS
C = 128
L = 16


def _gather_kernel(cpt, hpad, xt4, etab, srcp, dstp, hs, hd, dir4,
                   xt4v, etv, isrc, idst, bsh, bdh, dbuf, sem):
    wid = lax.axis_index("s") * NC + lax.axis_index("c")
    base0 = wid * (C * cpt)
    pltpu.sync_copy(xt4, xt4v)
    pltpu.sync_copy(etab, etv)

    def body(j, _):
        base = base0 + j * C
        pltpu.sync_copy(srcp.at[pl.ds(base, C)], isrc)
        pltpu.sync_copy(dstp.at[pl.ds(base, C)], idst)
        c1 = pltpu.async_copy(hpad.at[isrc], bsh, sem)
        c2 = pltpu.async_copy(hpad.at[idst], bdh, sem)

        def xbody(k, _):
            ev = etv[pl.ds(L * k, L)]
            comp = etv[pl.ds((C // 4) * L, L)]
            s4 = plsc.load_gather(isrc, [ev]) * 4 + comp
            d4 = plsc.load_gather(idst, [ev]) * 4 + comp
            dv = plsc.load_gather(xt4v, [s4]) - plsc.load_gather(xt4v, [d4])
            dbuf[pl.ds(L * k, L)] = dv
            return ()

        lax.fori_loop(0, C // 4, xbody, ())
        c1.wait()
        c2.wait()
        pltpu.sync_copy(bsh, hs.at[pl.ds(base, C)])
        pltpu.sync_copy(bdh, hd.at[pl.ds(base, C)])
        pltpu.sync_copy(dbuf, dir4.at[pl.ds(base * 4, C * 4)])
        return ()

    lax.fori_loop(0, cpt, body, ())


def _scatter_kernel(cpt2, npad, m, cuf, dstp, zh, ph, pxt,
                    acc_h, accx, bm, bcf, idst, sem):
    cid = lax.axis_index("c")
    sid = lax.axis_index("s")
    wid = sid * NC + cid

    rpt = npad // NS
    pltpu.sync_copy(zh.at[pl.ds(sid * rpt, rpt)],
                    acc_h.at[pl.ds(sid * rpt, rpt)])

    def zbody(i, _):
        accx[pl.ds(L * i, L)] = jnp.zeros((L,), jnp.float32)
        return ()

    lax.fori_loop(0, npad * 3 // L, zbody, ())
    plsc.subcore_barrier()

    base0 = wid * (C * cpt2)

    def body(j, _):
        base = base0 + j * C
        pltpu.sync_copy(dstp.at[pl.ds(base, C)], idst)
        pltpu.sync_copy(m.at[pl.ds(base, C)], bm)
        cpy = pltpu.async_copy(bm, acc_h.at[idst], sem, add=True)
        pltpu.sync_copy(cuf.at[pl.ds(base * L, C * L)], bcf)

        def xbody(e, _):
            ev = jnp.full((L,), e, jnp.int32)
            dvec = plsc.load_gather(idst, [ev])
            addr = dvec * 3 + lax.iota(jnp.int32, L)
            vals = bcf[pl.ds(L * e, L)]
            plsc.addupdate_scatter(accx, [addr], vals)
            return ()

        lax.fori_loop(0, C, xbody, ())
        cpy.wait()
        return ()

    lax.fori_loop(0, cpt2, body, ())
    plsc.subcore_barrier()

    @pl.when(sid == 0)
    def _():
        pltpu.sync_copy(acc_h, ph.at[cid])

    pltpu.sync_copy(accx, pxt.at[wid])


def _edge_kernel(hs_ref, hd_ref, dir_ref, dist_ref,
                 we1_ref, be1_ref, we2_ref, be2_ref,
                 w1s_ref, w1d_ref, w1e_ref, b1_ref,
                 wn2_ref, bn2_ref, wc2_ref,
                 m_ref, cu_ref):
    f32 = jnp.float32
    hp = None
    d = dist_ref[0, 0, :][:, None]
    ea = d * we1_ref[...] + be1_ref[...]
    ea = ea * jax.nn.sigmoid(ea)
    ea = jnp.dot(ea, we2_ref[...], precision=hp, preferred_element_type=f32)
    ea = ea + be2_ref[...]
    z = jnp.dot(hs_ref[...], w1s_ref[...], precision=hp, preferred_element_type=f32)
    z = z + jnp.dot(hd_ref[...], w1d_ref[...], precision=hp, preferred_element_type=f32)
    z = z + jnp.dot(ea, w1e_ref[...], precision=hp, preferred_element_type=f32)
    z = z + b1_ref[...]
    u = z * jax.nn.sigmoid(z)
    un = u[:, :128]
    uc = u[:, 128:]
    m = jnp.dot(un, wn2_ref[...], precision=hp, preferred_element_type=f32)
    m_ref[...] = m + bn2_ref[...]
    cw = jnp.sum(uc * wc2_ref[...], axis=1, keepdims=True)
    dirv = dir_ref[...]
    n2 = jnp.sum(dirv * dirv, axis=1, keepdims=True)
    dl = jnp.maximum(jnp.sqrt(n2), 1e-8)
    cu4 = (cw / dl) * dirv
    cu_ref[...] = jnp.concatenate(
        [cu4, jnp.zeros((cu4.shape[0], 12), f32)], axis=1)


def _combine_kernel(hpad_ref, xpp_ref, ph_ref, px_ref, ho_ref, xo_ref):
    ho_ref[...] = hpad_ref[...] + ph_ref[0] + ph_ref[1]
    xo_ref[...] = xpp_ref[...] + jnp.sum(px_ref[...], axis=0)


def kernel(h, x, edge_index, edge_dist, We1, be1, We2, be2,
           Wn1, bn1, Wn2, bn2, Wc1, bc1, Wc2):
    f32 = jnp.float32
    i32 = jnp.int32
    N, D = h.shape
    E = edge_index.shape[1]

    cpt = -(-E // (NW * C))
    e_pad = NW * C * cpt
    cpt2 = e_pad // (NW * C)
    npad = -(-(N + 1) // 128) * 128

    src = jnp.pad(edge_index[0], (0, e_pad - E))
    dst = jnp.pad(edge_index[1], (0, e_pad - E),
                  constant_values=N)
    distp = jnp.pad(edge_dist, (0, e_pad - E))

    hpad = jnp.pad(h, ((0, npad - N), (0, 0)))
    xt4 = jnp.pad(x, ((0, npad - N), (0, 4 - x.shape[1]))).reshape(-1)
    xpp3 = x if x.shape[1] == 3 else jnp.pad(x, ((0, 0), (0, 3 - x.shape[1])))
    xpp3 = jnp.pad(xpp3, ((0, npad - N), (0, 0)))

    p = jnp.arange(C // 4 * L, dtype=i32)
    etab = (p // L * 4 + p % L // 4).reshape(C // 4, L)
    etab = jnp.concatenate(
        [etab, jnp.tile(jnp.arange(4, dtype=i32), L // 4)[None, :]]).reshape(-1)

    W1 = jnp.concatenate([Wn1, Wc1], axis=1)
    W1s, W1d, W1e = W1[:D], W1[D:2 * D], W1[2 * D:]
    b1 = jnp.concatenate([bn1, bc1])[None, :]
    wc2 = Wc2[:, 0][None, :]

    mesh = plsc.VectorSubcoreMesh(core_axis_name="c", subcore_axis_name="s",
                                  num_cores=NC, num_subcores=NS)

    gather = pl.kernel(
        functools.partial(_gather_kernel, cpt),
        out_type=(
            jax.ShapeDtypeStruct((e_pad, D), f32),
            jax.ShapeDtypeStruct((e_pad, D), f32),
            jax.ShapeDtypeStruct((e_pad * 4,), f32),
        ),
        mesh=mesh,
        scratch_types=[
            pltpu.VMEM((npad * 4,), f32),
            pltpu.VMEM(((C // 4 + 1) * L,), i32),
            pltpu.VMEM((C,), i32),
            pltpu.VMEM((C,), i32),
            pltpu.VMEM((C, D), f32),
            pltpu.VMEM((C, D), f32),
            pltpu.VMEM((C * 4,), f32),
            pltpu.SemaphoreType.DMA,
        ],
        name="egnn_sc_gather",
        compiler_params=pltpu.CompilerParams(needs_layout_passes=False),
    )
    hs, hd, dir4 = gather(hpad, xt4, etab, src, dst)

    BE = 512
    grid = e_pad // BE
    dist3 = distp.reshape(grid, 1, BE)
    full = lambda shape: pl.BlockSpec(shape, lambda i: (0,) * len(shape))
    m, cu = pl.pallas_call(
        _edge_kernel,
        grid=(grid,),
        in_specs=[
            pl.BlockSpec((BE, D), lambda i: (i, 0)),
            pl.BlockSpec((BE, D), lambda i: (i, 0)),
            pl.BlockSpec((BE, 4), lambda i: (i, 0)),
            pl.BlockSpec((1, 1, BE), lambda i: (i, 0, 0)),
            full((1, 16)), full((1, 16)), full((16, 16)), full((1, 16)),
            full((D, 256)), full((D, 256)), full((16, 256)), full((1, 256)),
            full((D, D)), full((1, D)), full((1, D)),
        ],
        out_specs=[
            pl.BlockSpec((BE, D), lambda i: (i, 0)),
            pl.BlockSpec((BE, 16), lambda i: (i, 0)),
        ],
        out_shape=[
            jax.ShapeDtypeStruct((e_pad, D), f32),
            jax.ShapeDtypeStruct((e_pad, 16), f32),
        ],
        name="egnn_tc_edge_mlp",
    )(hs, hd, dir4.reshape(e_pad, 4), dist3,
      We1, be1[None, :], We2, be2[None, :],
      W1s, W1d, W1e, b1, Wn2, bn2[None, :], wc2)

    zh = jnp.zeros((npad, D), f32)
    scatter = pl.kernel(
        functools.partial(_scatter_kernel, cpt2, npad),
        out_type=(
            jax.ShapeDtypeStruct((NC, npad, D), f32),
            jax.ShapeDtypeStruct((NW, npad * 3), f32),
        ),
        mesh=mesh,
        scratch_types=[
            pltpu.VMEM_SHARED((npad, D), f32),
            pltpu.VMEM((npad * 3,), f32),
            pltpu.VMEM((C, D), f32),
            pltpu.VMEM((C * L,), f32),
            pltpu.VMEM((C,), i32),
            pltpu.SemaphoreType.DMA,
        ],
        name="egnn_sc_scatter",
        compiler_params=pltpu.CompilerParams(needs_layout_passes=False),
    )
    ph, pxt = scatter(m, cu.reshape(-1), dst, zh)

    RB = 128
    ho, xo = pl.pallas_call(
        _combine_kernel,
        grid=(npad // RB,),
        in_specs=[
            pl.BlockSpec((RB, D), lambda i: (i, 0)),
            pl.BlockSpec((RB, 3), lambda i: (i, 0)),
            pl.BlockSpec((NC, RB, D), lambda i: (0, i, 0)),
            pl.BlockSpec((NW, RB, 3), lambda i: (0, i, 0)),
        ],
        out_specs=[
            pl.BlockSpec((RB, D), lambda i: (i, 0)),
            pl.BlockSpec((RB, 3), lambda i: (i, 0)),
        ],
        out_shape=[
            jax.ShapeDtypeStruct((npad, D), f32),
            jax.ShapeDtypeStruct((npad, 3), f32),
        ],
        name="egnn_tc_combine",
    )(hpad, xpp3, ph, pxt.reshape(NW, npad, 3))

    return ho[:N], xo[:N]
